# baseline (speedup 1.0000x reference)
import functools

import jax
import jax.numpy as jnp
from jax import lax
from jax.experimental import pallas as pl
from jax.experimental.pallas import tpu as pltpu
from jax.experimental.pallas import tpu_sc as plsc

NUM_EMB = 1000000
DIM = 32
NROW = 16384
NCOL = 26

NC = 2
NS = 16
NW = NC * NS

RPT = NROW // NW
R = RPT * NCOL
CR = 32
CH = CR * NCOL
NCHUNK = RPT // CR
assert NCHUNK * CR == RPT


def _body(idx_hbm, table_hbm, out_hbm, idx_v, buf0, buf1,
          gsem0, gsem1, ssem0, ssem1):
    wid = lax.axis_index("s") * NC + lax.axis_index("c")
    b0 = wid * RPT
    pltpu.sync_copy(idx_hbm.at[wid], idx_v)

    bufs = (buf0, buf1)
    gsems = (gsem0, gsem1)
    ssems = (ssem0, ssem1)
    g = [None, None]

    def _drain_scatters(b):
        pltpu.make_async_copy(
            table_hbm.at[pl.ds(0, CH)], bufs[b], ssems[b]).wait()

    g[0] = pltpu.async_copy(
        table_hbm.at[idx_v.at[pl.ds(0, CH)]], bufs[0], gsems[0])
    for c in range(NCHUNK):
        b = c & 1
        nb = b ^ 1
        if c + 1 < NCHUNK:
            if c >= 1:
                _drain_scatters(nb)
            g[nb] = pltpu.async_copy(
                table_hbm.at[idx_v.at[pl.ds((c + 1) * CH, CH)]],
                bufs[nb], gsems[nb])
        g[b].wait()

        def _scatter_row(r, carry, _b=b, _c=c):
            pltpu.async_copy(
                bufs[_b].at[pl.ds(r * NCOL, NCOL)],
                out_hbm.at[b0 + _c * CR + r], ssems[_b])
            return carry
        lax.fori_loop(0, CR, _scatter_row, 0)

    _drain_scatters((NCHUNK - 2) & 1)
    _drain_scatters((NCHUNK - 1) & 1)


def _gather(idx, table):
    mesh = plsc.VectorSubcoreMesh(core_axis_name="c", subcore_axis_name="s")
    k = pl.kernel(
        _body,
        mesh=mesh,
        compiler_params=pltpu.CompilerParams(use_tc_tiling_on_sc=False),
        out_type=jax.ShapeDtypeStruct((NROW, NCOL, DIM), jnp.float32),
        scratch_types=[
            pltpu.VMEM((R,), jnp.int32),
            pltpu.VMEM((CH, DIM), jnp.float32),
            pltpu.VMEM((CH, DIM), jnp.float32),
            pltpu.SemaphoreType.DMA,
            pltpu.SemaphoreType.DMA,
            pltpu.SemaphoreType.DMA,
            pltpu.SemaphoreType.DMA,
        ],
    )
    return k(idx, table)


def kernel(inputs, weight):
    idx = inputs.astype(jnp.int32).reshape(NW, R)
    return _gather(idx, weight)
